# baseline (speedup 1.0000x reference)
import functools

import jax
import jax.numpy as jnp
from jax import lax
from jax.experimental import pallas as pl
from jax.experimental.pallas import tpu as pltpu
from jax.experimental.pallas import tpu_sc as plsc

N = 10000
E = 320000
D = 128

NC = 2
NS = 16
NW = NC * NS

NPAD = 10240
HALF = NPAD // NC
TRASH = HALF
ACC_ROWS = HALF + 1
RPT = HALF // NS

CH = 160
EPT = CH * 128
EPAD = NS * EPT

BR = 1024
NBLK = NPAD // BR

_f32 = jnp.float32


def _deg_body(dst_hbm, out_hbm, dst_v, buf_v, acc, sem):
    c = lax.axis_index("c")
    s = lax.axis_index("s")

    def _zero(i, carry):
        buf_v[pl.ds(i * 16, 16)] = jnp.zeros((16,), _f32)
        return carry

    lax.fori_loop(0, RPT // 16, _zero, 0)
    pltpu.sync_copy(buf_v, acc.at[pl.ds(s * RPT, RPT)])
    plsc.subcore_barrier()

    def _ones(i, carry):
        buf_v[pl.ds(i * 16, 16)] = jnp.ones((16,), _f32)
        return carry

    lax.fori_loop(0, 128 // 16, _ones, 0)
    pltpu.sync_copy(dst_hbm.at[c, s], dst_v)

    def _count(j, carry):
        pltpu.async_copy(buf_v.at[pl.ds(0, 128)], acc.at[dst_v.at[j]], sem,
                         add=True)

        @pl.when(j >= 8)
        def _():
            pltpu.make_async_copy(buf_v.at[pl.ds(0, 128)],
                                  acc.at[dst_v.at[0]], sem).wait()

        return carry

    lax.fori_loop(0, CH, _count, 0)
    for _ in range(8):
        pltpu.make_async_copy(buf_v.at[pl.ds(0, 128)], acc.at[dst_v.at[0]],
                              sem).wait()
    plsc.subcore_barrier()
    pltpu.sync_copy(acc.at[pl.ds(s * RPT, RPT)], out_hbm.at[c, s])


NBUF = 2
NT = CH // NBUF


def _agg_body(hp_hbm, src_hbm, dst_hbm, out_hbm, src_v, dst_v, *scratch):
    rbuf = scratch[:NBUF]
    acc = scratch[NBUF]
    gs = scratch[NBUF + 1:NBUF + 1 + NBUF]
    ss = scratch[NBUF + 1 + NBUF:]
    c = lax.axis_index("c")
    s = lax.axis_index("s")

    def _zero(i, carry):
        rbuf[0][i // 8, pl.ds((i % 8) * 16, 16)] = jnp.zeros((16,), _f32)
        return carry

    lax.fori_loop(0, 128 * 8, _zero, 0)
    for k in range(RPT // 128):
        pltpu.sync_copy(rbuf[0], acc.at[pl.ds(s * RPT + k * 128, 128)])
    pltpu.sync_copy(rbuf[0].at[pl.ds(0, RPT % 128)],
                    acc.at[pl.ds(s * RPT + (RPT // 128) * 128, RPT % 128)])
    plsc.subcore_barrier()

    pltpu.sync_copy(src_hbm.at[s], src_v)
    pltpu.sync_copy(dst_hbm.at[c, s], dst_v)

    for b in range(NBUF):
        pltpu.async_copy(hp_hbm.at[src_v.at[b]], rbuf[b], gs[b])

    def _body(t, carry):
        for b in range(NBUF):
            j = t * NBUF + b
            pltpu.make_async_copy(hp_hbm.at[src_v.at[j]], rbuf[b],
                                  gs[b]).wait()
            pltpu.async_copy(rbuf[b], acc.at[dst_v.at[j]], ss[b], add=True)
        for b in range(NBUF):
            j = t * NBUF + b

            @pl.when(t < NT - 1)
            def _():
                pltpu.make_async_copy(rbuf[b], acc.at[dst_v.at[j]],
                                      ss[b]).wait()
                pltpu.async_copy(hp_hbm.at[src_v.at[j + NBUF]], rbuf[b],
                                 gs[b])

        return carry

    lax.fori_loop(0, NT, _body, 0)
    for b in range(NBUF):
        j = (NT - 1) * NBUF + b
        pltpu.make_async_copy(rbuf[b], acc.at[dst_v.at[j]], ss[b]).wait()
    plsc.subcore_barrier()
    pltpu.sync_copy(acc.at[pl.ds(s * RPT, RPT)], out_hbm.at[c, s])


@functools.cache
def _sc_kernels():
    mesh = plsc.VectorSubcoreMesh(core_axis_name="c", subcore_axis_name="s",
                                  num_cores=NC, num_subcores=NS)
    params = pltpu.CompilerParams(use_tc_tiling_on_sc=False)
    deg = pl.kernel(
        _deg_body,
        out_type=jax.ShapeDtypeStruct((NC, NS, RPT), _f32),
        mesh=mesh,
        scratch_types=[
            pltpu.VMEM((CH, 128), jnp.int32),
            pltpu.VMEM((RPT,), _f32),
            pltpu.VMEM_SHARED((ACC_ROWS,), _f32),
            pltpu.SemaphoreType.DMA,
        ],
        compiler_params=params,
    )
    agg = pl.kernel(
        _agg_body,
        out_type=jax.ShapeDtypeStruct((NC, NS, RPT, D), _f32),
        mesh=mesh,
        scratch_types=[
            pltpu.VMEM((CH, 128), jnp.int32),
            pltpu.VMEM((CH, 128), jnp.int32),
        ] + [pltpu.VMEM((128, D), _f32)] * NBUF + [
            pltpu.VMEM_SHARED((ACC_ROWS, D), _f32),
        ] + [pltpu.SemaphoreType.DMA] * (2 * NBUF),
        compiler_params=params,
    )
    return deg, agg


def _mm_scale_body(x_ref, w_ref, d_ref, o_ref):
    o_ref[...] = jnp.dot(x_ref[...], w_ref[...],
                         preferred_element_type=_f32) * d_ref[...]


_mm_scale = pl.pallas_call(
    _mm_scale_body,
    grid=(NBLK,),
    in_specs=[
        pl.BlockSpec((BR, D), lambda i: (i, 0)),
        pl.BlockSpec((D, D), lambda i: (0, 0)),
        pl.BlockSpec((BR, D), lambda i: (i, 0)),
    ],
    out_specs=pl.BlockSpec((BR, D), lambda i: (i, 0)),
    out_shape=jax.ShapeDtypeStruct((NPAD, D), _f32),
)


def _layer_body(s_ref, hp_ref, d_ref, b_ref, w_ref, o_ref):
    t = (s_ref[...] + hp_ref[...]) * d_ref[...] + b_ref[...]
    h = jnp.maximum(t, 0.0)
    o_ref[...] = jnp.dot(h, w_ref[...], preferred_element_type=_f32) * d_ref[...]


_layer = pl.pallas_call(
    _layer_body,
    grid=(NBLK,),
    in_specs=[
        pl.BlockSpec((BR, D), lambda i: (i, 0)),
        pl.BlockSpec((BR, D), lambda i: (i, 0)),
        pl.BlockSpec((BR, D), lambda i: (i, 0)),
        pl.BlockSpec((1, D), lambda i: (0, 0)),
        pl.BlockSpec((D, D), lambda i: (0, 0)),
    ],
    out_specs=pl.BlockSpec((BR, D), lambda i: (i, 0)),
    out_shape=jax.ShapeDtypeStruct((NPAD, D), _f32),
)


def _final_body(s_ref, hp_ref, d_ref, b_ref, o_ref):
    i = pl.program_id(0)
    t = (s_ref[...] + hp_ref[...]) * d_ref[...] + b_ref[...]
    h = jnp.maximum(t, 0.0)
    row = lax.broadcasted_iota(jnp.int32, (BR, D), 0) + i * BR
    h = jnp.where(row < N, h, 0.0)

    @pl.when(i == 0)
    def _():
        o_ref[...] = jnp.zeros_like(o_ref)

    o_ref[...] += jnp.sum(h, axis=0, keepdims=True)


_final = pl.pallas_call(
    _final_body,
    grid=(NBLK,),
    in_specs=[
        pl.BlockSpec((BR, D), lambda i: (i, 0)),
        pl.BlockSpec((BR, D), lambda i: (i, 0)),
        pl.BlockSpec((BR, D), lambda i: (i, 0)),
        pl.BlockSpec((1, D), lambda i: (0, 0)),
    ],
    out_specs=pl.BlockSpec((1, D), lambda i: (0, 0)),
    out_shape=jax.ShapeDtypeStruct((1, D), _f32),
)


def kernel(x, edge_index, W1, b1, W2, b2, Wlin, blin):
    src = edge_index[0].astype(jnp.int32)
    dst = edge_index[1].astype(jnp.int32)
    pad_t = EPT - E // NS
    src2 = src.reshape(NS, E // NS)
    srcp = jnp.concatenate(
        [src2, jnp.zeros((NS, pad_t), jnp.int32)], axis=1).reshape(NS, CH, 128)
    dst2 = dst.reshape(NS, E // NS)
    dst2 = jnp.concatenate(
        [dst2, jnp.full((NS, pad_t), -1, jnp.int32)], axis=1)
    halves = []
    for c in range(NC):
        local = dst2 - c * HALF
        ok = (local >= 0) & (local < HALF)
        halves.append(jnp.where(ok, local, TRASH))
    dstp = jnp.stack(halves).reshape(NC, NS, CH, 128)
    xp = jnp.concatenate(
        [x.astype(_f32), jnp.zeros((NPAD - N, D), _f32)], axis=0)

    _deg_kernel, _agg_kernel = _sc_kernels()
    cnt = _deg_kernel(dstp).reshape(NPAD)
    dinv = lax.rsqrt(cnt + 1.0)
    dinv_b = jnp.broadcast_to(dinv[:, None], (NPAD, D))

    h1p = _mm_scale(xp, W1.astype(_f32), dinv_b)
    S1 = _agg_kernel(h1p, srcp, dstp).reshape(NPAD, D)
    h2p = _layer(S1, h1p, dinv_b, b1.reshape(1, D).astype(_f32),
                 W2.astype(_f32))
    S2 = _agg_kernel(h2p, srcp, dstp).reshape(NPAD, D)
    gsum = _final(S2, h2p, dinv_b, b2.reshape(1, D).astype(_f32))

    out = (gsum * (1.0 / N)) @ Wlin.astype(_f32) + blin.astype(_f32)
    return out


# speedup vs baseline: 1.0357x; 1.0357x over previous
import functools

import jax
import jax.numpy as jnp
from jax import lax
from jax.experimental import pallas as pl
from jax.experimental.pallas import tpu as pltpu
from jax.experimental.pallas import tpu_sc as plsc

N = 10000
E = 320000
D = 128

NC = 2
NS = 16
NW = NC * NS

NPAD = 10240
HALF = NPAD // NC
TRASH = HALF
ACC_ROWS = HALF + 8
RPT = HALF // NS

CH = 160
EPT = CH * 128
EPAD = NS * EPT

BR = 1024
NBLK = NPAD // BR

_f32 = jnp.float32


def _deg_body(dst_hbm, out_hbm, dst_v, buf_v, acc, sem):
    c = lax.axis_index("c")
    s = lax.axis_index("s")

    def _zero(i, carry):
        buf_v[pl.ds(i * 16, 16)] = jnp.zeros((16,), _f32)
        return carry

    lax.fori_loop(0, RPT // 16, _zero, 0)
    pltpu.sync_copy(buf_v, acc.at[pl.ds(s * RPT, RPT)])
    plsc.subcore_barrier()

    def _ones(i, carry):
        buf_v[pl.ds(i * 16, 16)] = jnp.ones((16,), _f32)
        return carry

    lax.fori_loop(0, 128 // 16, _ones, 0)
    pltpu.sync_copy(dst_hbm.at[c, s], dst_v)

    def _count(j, carry):
        pltpu.async_copy(buf_v.at[pl.ds(0, 128)], acc.at[dst_v.at[j]], sem,
                         add=True)

        @pl.when(j >= 8)
        def _():
            pltpu.make_async_copy(buf_v.at[pl.ds(0, 128)],
                                  acc.at[dst_v.at[0]], sem).wait()

        return carry

    lax.fori_loop(0, CH, _count, 0)
    for _ in range(8):
        pltpu.make_async_copy(buf_v.at[pl.ds(0, 128)], acc.at[dst_v.at[0]],
                              sem).wait()
    plsc.subcore_barrier()
    pltpu.sync_copy(acc.at[pl.ds(s * RPT, RPT)], out_hbm.at[c, s])


def _agg_body(hp_hbm, src_hbm, dst_hbm, out_hbm, src_v, dst_v, ra, rb, acc,
              sa, sb):
    c = lax.axis_index("c")
    s = lax.axis_index("s")

    def _zero(i, carry):
        ra[i // 8, pl.ds((i % 8) * 16, 16)] = jnp.zeros((16,), _f32)
        return carry

    lax.fori_loop(0, 128 * 8, _zero, 0)
    for k in range(RPT // 128):
        pltpu.sync_copy(ra, acc.at[pl.ds(s * RPT + k * 128, 128)])
    pltpu.sync_copy(ra.at[pl.ds(0, RPT % 128)],
                    acc.at[pl.ds(s * RPT + (RPT // 128) * 128, RPT % 128)])
    plsc.subcore_barrier()

    pltpu.sync_copy(src_hbm.at[s], src_v)
    pltpu.sync_copy(dst_hbm.at[c, s], dst_v)

    pltpu.async_copy(hp_hbm.at[src_v.at[0]], ra, sa)

    def _body(t, carry):
        j0 = 2 * t
        j1 = j0 + 1
        pltpu.make_async_copy(hp_hbm.at[src_v.at[j0]], ra, sa).wait()
        pltpu.async_copy(hp_hbm.at[src_v.at[j1]], rb, sb)
        pltpu.sync_copy(ra, acc.at[dst_v.at[j0]], add=True)
        pltpu.make_async_copy(hp_hbm.at[src_v.at[j1]], rb, sb).wait()

        @pl.when(t < CH // 2 - 1)
        def _():
            pltpu.async_copy(hp_hbm.at[src_v.at[j0 + 2]], ra, sa)

        pltpu.sync_copy(rb, acc.at[dst_v.at[j1]], add=True)
        return carry

    lax.fori_loop(0, CH // 2, _body, 0)
    plsc.subcore_barrier()
    pltpu.sync_copy(acc.at[pl.ds(s * RPT, RPT)], out_hbm.at[c, s])


@functools.cache
def _sc_kernels():
    mesh = plsc.VectorSubcoreMesh(core_axis_name="c", subcore_axis_name="s",
                                  num_cores=NC, num_subcores=NS)
    params = pltpu.CompilerParams(use_tc_tiling_on_sc=False)
    deg = pl.kernel(
        _deg_body,
        out_type=jax.ShapeDtypeStruct((NC, NS, RPT), _f32),
        mesh=mesh,
        scratch_types=[
            pltpu.VMEM((CH, 128), jnp.int32),
            pltpu.VMEM((RPT,), _f32),
            pltpu.VMEM_SHARED((ACC_ROWS,), _f32),
            pltpu.SemaphoreType.DMA,
        ],
        compiler_params=params,
    )
    agg = pl.kernel(
        _agg_body,
        out_type=jax.ShapeDtypeStruct((NC, NS, RPT, D), _f32),
        mesh=mesh,
        scratch_types=[
            pltpu.VMEM((CH, 128), jnp.int32),
            pltpu.VMEM((CH, 128), jnp.int32),
            pltpu.VMEM((128, D), _f32),
            pltpu.VMEM((128, D), _f32),
            pltpu.VMEM_SHARED((ACC_ROWS, D), _f32),
            pltpu.SemaphoreType.DMA,
            pltpu.SemaphoreType.DMA,
        ],
        compiler_params=params,
    )
    return deg, agg


def _mm_scale_body(x_ref, w_ref, d_ref, o_ref):
    o_ref[...] = jnp.dot(x_ref[...], w_ref[...],
                         preferred_element_type=_f32) * d_ref[...]


_mm_scale = pl.pallas_call(
    _mm_scale_body,
    grid=(NBLK,),
    in_specs=[
        pl.BlockSpec((BR, D), lambda i: (i, 0)),
        pl.BlockSpec((D, D), lambda i: (0, 0)),
        pl.BlockSpec((BR, D), lambda i: (i, 0)),
    ],
    out_specs=pl.BlockSpec((BR, D), lambda i: (i, 0)),
    out_shape=jax.ShapeDtypeStruct((NPAD, D), _f32),
)


def _layer_body(s_ref, hp_ref, d_ref, b_ref, w_ref, o_ref):
    t = (s_ref[...] + hp_ref[...]) * d_ref[...] + b_ref[...]
    h = jnp.maximum(t, 0.0)
    o_ref[...] = jnp.dot(h, w_ref[...], preferred_element_type=_f32) * d_ref[...]


_layer = pl.pallas_call(
    _layer_body,
    grid=(NBLK,),
    in_specs=[
        pl.BlockSpec((BR, D), lambda i: (i, 0)),
        pl.BlockSpec((BR, D), lambda i: (i, 0)),
        pl.BlockSpec((BR, D), lambda i: (i, 0)),
        pl.BlockSpec((1, D), lambda i: (0, 0)),
        pl.BlockSpec((D, D), lambda i: (0, 0)),
    ],
    out_specs=pl.BlockSpec((BR, D), lambda i: (i, 0)),
    out_shape=jax.ShapeDtypeStruct((NPAD, D), _f32),
)


def _final_body(s_ref, hp_ref, d_ref, b_ref, o_ref):
    i = pl.program_id(0)
    t = (s_ref[...] + hp_ref[...]) * d_ref[...] + b_ref[...]
    h = jnp.maximum(t, 0.0)
    row = lax.broadcasted_iota(jnp.int32, (BR, D), 0) + i * BR
    h = jnp.where(row < N, h, 0.0)

    @pl.when(i == 0)
    def _():
        o_ref[...] = jnp.zeros_like(o_ref)

    o_ref[...] += jnp.sum(h, axis=0, keepdims=True)


_final = pl.pallas_call(
    _final_body,
    grid=(NBLK,),
    in_specs=[
        pl.BlockSpec((BR, D), lambda i: (i, 0)),
        pl.BlockSpec((BR, D), lambda i: (i, 0)),
        pl.BlockSpec((BR, D), lambda i: (i, 0)),
        pl.BlockSpec((1, D), lambda i: (0, 0)),
    ],
    out_specs=pl.BlockSpec((1, D), lambda i: (0, 0)),
    out_shape=jax.ShapeDtypeStruct((1, D), _f32),
)


def kernel(x, edge_index, W1, b1, W2, b2, Wlin, blin):
    src = edge_index[0].astype(jnp.int32)
    dst = edge_index[1].astype(jnp.int32)
    pad_t = EPT - E // NS
    src2 = src.reshape(NS, E // NS)
    srcp = jnp.concatenate(
        [src2, jnp.zeros((NS, pad_t), jnp.int32)], axis=1).reshape(NS, CH, 128)
    dst2 = dst.reshape(NS, E // NS)
    dst2 = jnp.concatenate(
        [dst2, jnp.full((NS, pad_t), -1, jnp.int32)], axis=1)
    halves = []
    for c in range(NC):
        local = dst2 - c * HALF
        ok = (local >= 0) & (local < HALF)
        halves.append(jnp.where(ok, local, TRASH))
    dstp = jnp.stack(halves).reshape(NC, NS, CH, 128)
    xp = jnp.concatenate(
        [x.astype(_f32), jnp.zeros((NPAD - N, D), _f32)], axis=0)

    _deg_kernel, _agg_kernel = _sc_kernels()
    cnt = _deg_kernel(dstp).reshape(NPAD)
    dinv = lax.rsqrt(cnt + 1.0)
    dinv_b = jnp.broadcast_to(dinv[:, None], (NPAD, D))

    h1p = _mm_scale(xp, W1.astype(_f32), dinv_b)
    S1 = _agg_kernel(h1p, srcp, dstp).reshape(NPAD, D)
    h2p = _layer(S1, h1p, dinv_b, b1.reshape(1, D).astype(_f32),
                 W2.astype(_f32))
    S2 = _agg_kernel(h2p, srcp, dstp).reshape(NPAD, D)
    gsum = _final(S2, h2p, dinv_b, b2.reshape(1, D).astype(_f32))

    out = (gsum * (1.0 / N)) @ Wlin.astype(_f32) + blin.astype(_f32)
    return out
